# baseline (speedup 1.0000x reference)
import functools

import jax
import jax.numpy as jnp
from jax import lax
from jax.experimental import pallas as pl
from jax.experimental.pallas import tpu as pltpu
from jax.experimental.pallas import tpu_sc as plsc

_N = 100000
_E = 3200000
_CH = 256
_GROUP = 4
_NW = 32
_NGROUPS = _E // (_CH * _GROUP)
_NCHUNKS = _E // _CH
_NA = _N + 96
_CW = 6256
_NP = _NA // 8
_BP = 3128
_GRIDP = _NP // _BP


def _sc_agg_body(e3_hbm, h_hbm, z16_hbm, z1_hbm, ssum_out, cnt_out,
                 sidx, didx, rbuf, ones,
                 ssum_sh, cnt_sh,
                 g0, g1, g2, g3, g4, g5, g6, g7, ssem, csem, isem):
    cid = lax.axis_index("c")
    sid = lax.axis_index("s")
    wid = cid * 16 + sid
    gsems = (g0, g1, g2, g3, g4, g5, g6, g7)

    one16 = jnp.ones((16,), jnp.float32)

    def ofill(i, c):
        ones[pl.ds(i * 16, 16)] = one16
        return c
    lax.fori_loop(0, _GROUP * _CH // 16, ofill, 0)

    rbase = sid * _CW
    pltpu.sync_copy(z16_hbm, ssum_sh.at[pl.ds(rbase, _CW)])
    pltpu.sync_copy(z1_hbm, cnt_sh.at[pl.ds(rbase, _CW)])

    plsc.subcore_barrier()

    _GE = _GROUP * _CH

    def fire_idx(b, g):
        pltpu.async_copy(e3_hbm.at[0, pl.ds(g * _GE, _GE)], sidx.at[b], isem)
        pltpu.async_copy(e3_hbm.at[1, pl.ds(g * _GE, _GE)], didx.at[b], isem)

    def wait_idx(b):
        pltpu.make_async_copy(e3_hbm.at[0, pl.ds(0, _GE)], sidx.at[b],
                              isem).wait()
        pltpu.make_async_copy(e3_hbm.at[1, pl.ds(0, _GE)], didx.at[b],
                              isem).wait()

    def drain_scatters(ob):
        for j in range(_GROUP):
            dj = didx.at[ob, pl.ds(j * _CH, _CH)]
            pltpu.make_async_copy(rbuf.at[pl.ds(j * _CH, _CH)],
                                  ssum_sh.at[dj], ssem).wait()
            pltpu.make_async_copy(ones.at[pl.ds(0, _CH)], cnt_sh.at[dj],
                                  csem).wait()

    def run_group(b):
        gd = [pltpu.async_copy(h_hbm.at[sidx.at[b, pl.ds(j * _CH, _CH)]],
                               rbuf.at[pl.ds(j * _CH, _CH)], gsems[j])
              for j in range(_GROUP)]
        for j in range(_GROUP):
            gd[j].wait()
            dj = didx.at[b, pl.ds(j * _CH, _CH)]
            pltpu.async_copy(rbuf.at[pl.ds(j * _CH, _CH)], ssum_sh.at[dj],
                             ssem, add=True)
            pltpu.async_copy(ones.at[pl.ds(0, _CH)], cnt_sh.at[dj], csem,
                             add=True)

    fire_idx(0, wid)

    def pair_body(k2, c):
        k0 = 2 * k2

        @pl.when(k2 > 0)
        def _():
            drain_scatters(1)
        wait_idx(0)
        fire_idx(1, wid + (k0 + 1) * _NW)
        run_group(0)

        drain_scatters(0)
        wait_idx(1)
        fire_idx(0, wid + (k0 + 2) * _NW)
        run_group(1)
        return c
    lax.fori_loop(0, 48, pair_body, 0)

    drain_scatters(1)
    wait_idx(0)

    @pl.when(wid < _NGROUPS - 97 * _NW)
    def _():
        fire_idx(1, wid + 97 * _NW)
    run_group(0)
    drain_scatters(0)

    @pl.when(wid < _NGROUPS - 97 * _NW)
    def _():
        wait_idx(1)
        run_group(1)
        drain_scatters(1)

    plsc.subcore_barrier()

    orow = cid * _NA + rbase
    pltpu.sync_copy(ssum_sh.at[pl.ds(rbase, _CW)],
                    ssum_out.at[pl.ds(orow, _CW)])
    pltpu.sync_copy(cnt_sh.at[pl.ds(rbase, _CW)],
                    cnt_out.at[pl.ds(orow, _CW)])


_sc_agg = functools.partial(
    pl.kernel,
    out_type=(jax.ShapeDtypeStruct((2 * _NA, 16), jnp.float32),
              jax.ShapeDtypeStruct((2 * _NA,), jnp.float32)),
    mesh=plsc.VectorSubcoreMesh(core_axis_name="c", subcore_axis_name="s"),
    compiler_params=pltpu.CompilerParams(use_tc_tiling_on_sc=False),
    scratch_types=[
        pltpu.VMEM((2, _GROUP * _CH), jnp.int32),
        pltpu.VMEM((2, _GROUP * _CH), jnp.int32),
        pltpu.VMEM((_GROUP * _CH, 16), jnp.float32),
        pltpu.VMEM((_GROUP * _CH,), jnp.float32),
        pltpu.VMEM_SHARED((_NA, 16), jnp.float32),
        pltpu.VMEM_SHARED((_NA,), jnp.float32),
    ] + [pltpu.SemaphoreType.DMA] * 11,
)(_sc_agg_body)


def _lin1_body(x_ref, w_ref, b_ref, o_ref):
    o_ref[...] = jnp.maximum(
        jnp.dot(x_ref[...], w_ref[...], preferred_element_type=jnp.float32)
        + b_ref[...], 0.0)


def _final_body(s0, s1, c0, c1, h_ref, wl, bl, wr, w2, b2, o_ref):
    cnt = jnp.maximum(c0[...] + c1[...], 1.0)
    aggr = (s0[...] + s1[...]) / cnt
    h2 = jnp.maximum(
        jnp.dot(aggr, wl[...], preferred_element_type=jnp.float32) + bl[...]
        + jnp.dot(h_ref[...], wr[...], preferred_element_type=jnp.float32),
        0.0)
    o_ref[...] = (jnp.dot(h2, w2[...], preferred_element_type=jnp.float32)
                  + b2[...])


@jax.jit
def kernel(x, edge_index, W1, b1, Wl, bl, Wr, W2, b2):
    eye8 = jnp.eye(8, dtype=jnp.float32)
    w1d = jnp.kron(eye8, W1.T)
    b1p = jnp.tile(b1, 8).reshape(1, 128)
    xp = jnp.pad(x.reshape(_N // 8, 128), ((0, (_NA - _N) // 8), (0, 0)))

    hp = pl.pallas_call(
        _lin1_body,
        grid=(_GRIDP,),
        in_specs=[pl.BlockSpec((_BP, 128), lambda i: (i, 0)),
                  pl.BlockSpec((128, 128), lambda i: (0, 0)),
                  pl.BlockSpec((1, 128), lambda i: (0, 0))],
        out_specs=pl.BlockSpec((_BP, 128), lambda i: (i, 0)),
        out_shape=jax.ShapeDtypeStruct((_NP, 128), jnp.float32),
    )(xp, w1d, b1p)

    h = hp.reshape(_NA, 16)
    e3 = edge_index
    z16 = jnp.zeros((_CW, 16), jnp.float32)
    z1 = jnp.zeros((_CW,), jnp.float32)
    ssum_p, cnt_p = _sc_agg(e3, h, z16, z1)

    sp = ssum_p.reshape(2 * _NP, 128)
    cexp = jnp.broadcast_to(cnt_p.reshape(2 * _NA, 1), (2 * _NA, 16))
    cp = cexp.reshape(2 * _NP, 128)

    wld = jnp.kron(eye8, Wl.T)
    wrd = jnp.kron(eye8, Wr.T)
    w2d = jnp.kron(eye8, W2.T)
    blp = jnp.tile(bl, 8).reshape(1, 128)
    b2p = jnp.tile(b2, 8).reshape(1, 256)

    outp = pl.pallas_call(
        _final_body,
        grid=(_GRIDP,),
        in_specs=[pl.BlockSpec((_BP, 128), lambda i: (i, 0)),
                  pl.BlockSpec((_BP, 128), lambda i: (i + _GRIDP, 0)),
                  pl.BlockSpec((_BP, 128), lambda i: (i, 0)),
                  pl.BlockSpec((_BP, 128), lambda i: (i + _GRIDP, 0)),
                  pl.BlockSpec((_BP, 128), lambda i: (i, 0)),
                  pl.BlockSpec((128, 128), lambda i: (0, 0)),
                  pl.BlockSpec((1, 128), lambda i: (0, 0)),
                  pl.BlockSpec((128, 128), lambda i: (0, 0)),
                  pl.BlockSpec((128, 256), lambda i: (0, 0)),
                  pl.BlockSpec((1, 256), lambda i: (0, 0))],
        out_specs=pl.BlockSpec((_BP, 256), lambda i: (i, 0)),
        out_shape=jax.ShapeDtypeStruct((_NP, 256), jnp.float32),
    )(sp, sp, cp, cp, hp, wld, blp, wrd, w2d, b2p)
    return outp[:_N // 8].reshape(_N, 32)


# speedup vs baseline: 1.0313x; 1.0313x over previous
import functools

import jax
import jax.numpy as jnp
from jax import lax
from jax.experimental import pallas as pl
from jax.experimental.pallas import tpu as pltpu
from jax.experimental.pallas import tpu_sc as plsc

_N = 100000
_E = 3200000
_CH = 128
_GROUP = 8
_NW = 32
_NGROUPS = _E // (_CH * _GROUP)
_NCHUNKS = _E // _CH
_NA = _N + 96
_CW = 6256
_NP = _NA // 8
_BP = 3128
_GRIDP = _NP // _BP


def _sc_agg_body(e3_hbm, h_hbm, z16_hbm, z1_hbm, ssum_out, cnt_out,
                 sidx, didx, rbuf, ones,
                 ssum_sh, cnt_sh,
                 g0, g1, g2, g3, g4, g5, g6, g7, ssem, csem, isem):
    cid = lax.axis_index("c")
    sid = lax.axis_index("s")
    wid = cid * 16 + sid
    gsems = (g0, g1, g2, g3, g4, g5, g6, g7)

    one16 = jnp.ones((16,), jnp.float32)

    def ofill(i, c):
        ones[pl.ds(i * 16, 16)] = one16
        return c
    lax.fori_loop(0, _GROUP * _CH // 16, ofill, 0)

    rbase = sid * _CW
    pltpu.sync_copy(z16_hbm, ssum_sh.at[pl.ds(rbase, _CW)])
    pltpu.sync_copy(z1_hbm, cnt_sh.at[pl.ds(rbase, _CW)])

    plsc.subcore_barrier()

    _GE = _GROUP * _CH

    def fire_idx(b, g):
        pltpu.async_copy(e3_hbm.at[pl.ds(g * _GE, _GE)], sidx.at[b], isem)
        pltpu.async_copy(e3_hbm.at[pl.ds(_E + g * _GE, _GE)], didx.at[b],
                         isem)

    def wait_idx(b):
        pltpu.make_async_copy(e3_hbm.at[pl.ds(0, _GE)], sidx.at[b],
                              isem).wait()
        pltpu.make_async_copy(e3_hbm.at[pl.ds(0, _GE)], didx.at[b],
                              isem).wait()

    def drain_scatters(ob):
        for j in range(_GROUP):
            dj = didx.at[ob, pl.ds(j * _CH, _CH)]
            pltpu.make_async_copy(rbuf.at[pl.ds(j * _CH, _CH)],
                                  ssum_sh.at[dj], ssem).wait()
            pltpu.make_async_copy(ones.at[pl.ds(0, _CH)], cnt_sh.at[dj],
                                  csem).wait()

    def run_group(b):
        gd = [pltpu.async_copy(h_hbm.at[sidx.at[b, pl.ds(j * _CH, _CH)]],
                               rbuf.at[pl.ds(j * _CH, _CH)], gsems[j])
              for j in range(_GROUP)]
        for j in range(_GROUP):
            gd[j].wait()
            dj = didx.at[b, pl.ds(j * _CH, _CH)]
            pltpu.async_copy(rbuf.at[pl.ds(j * _CH, _CH)], ssum_sh.at[dj],
                             ssem, add=True)
            pltpu.async_copy(ones.at[pl.ds(0, _CH)], cnt_sh.at[dj], csem,
                             add=True)

    fire_idx(0, wid)

    def pair_body(k2, c):
        k0 = 2 * k2

        @pl.when(k2 > 0)
        def _():
            drain_scatters(1)
        wait_idx(0)
        fire_idx(1, wid + (k0 + 1) * _NW)
        run_group(0)

        drain_scatters(0)
        wait_idx(1)
        fire_idx(0, wid + (k0 + 2) * _NW)
        run_group(1)
        return c
    lax.fori_loop(0, 48, pair_body, 0)

    drain_scatters(1)
    wait_idx(0)

    @pl.when(wid < _NGROUPS - 97 * _NW)
    def _():
        fire_idx(1, wid + 97 * _NW)
    run_group(0)
    drain_scatters(0)

    @pl.when(wid < _NGROUPS - 97 * _NW)
    def _():
        wait_idx(1)
        run_group(1)
        drain_scatters(1)

    plsc.subcore_barrier()

    orow = cid * _NA + rbase
    pltpu.sync_copy(ssum_sh.at[pl.ds(rbase, _CW)],
                    ssum_out.at[pl.ds(orow, _CW)])
    pltpu.sync_copy(cnt_sh.at[pl.ds(rbase, _CW)],
                    cnt_out.at[pl.ds(orow, _CW)])


_sc_agg = functools.partial(
    pl.kernel,
    out_type=(jax.ShapeDtypeStruct((2 * _NA, 16), jnp.float32),
              jax.ShapeDtypeStruct((2 * _NA,), jnp.float32)),
    mesh=plsc.VectorSubcoreMesh(core_axis_name="c", subcore_axis_name="s"),
    compiler_params=pltpu.CompilerParams(use_tc_tiling_on_sc=False),
    scratch_types=[
        pltpu.VMEM((2, _GROUP * _CH), jnp.int32),
        pltpu.VMEM((2, _GROUP * _CH), jnp.int32),
        pltpu.VMEM((_GROUP * _CH, 16), jnp.float32),
        pltpu.VMEM((_GROUP * _CH,), jnp.float32),
        pltpu.VMEM_SHARED((_NA, 16), jnp.float32),
        pltpu.VMEM_SHARED((_NA,), jnp.float32),
    ] + [pltpu.SemaphoreType.DMA] * 11,
)(_sc_agg_body)


def _lin1_body(x_ref, w_ref, b_ref, o_ref):
    o_ref[...] = jnp.maximum(
        jnp.dot(x_ref[...], w_ref[...], preferred_element_type=jnp.float32)
        + b_ref[...], 0.0)


def _final_body(s0, s1, c0, c1, h_ref, wl, bl, wr, w2, b2, o_ref):
    cnt = jnp.maximum(c0[...] + c1[...], 1.0)
    aggr = (s0[...] + s1[...]) / cnt
    h2 = jnp.maximum(
        jnp.dot(aggr, wl[...], preferred_element_type=jnp.float32) + bl[...]
        + jnp.dot(h_ref[...], wr[...], preferred_element_type=jnp.float32),
        0.0)
    o_ref[...] = (jnp.dot(h2, w2[...], preferred_element_type=jnp.float32)
                  + b2[...])


@jax.jit
def kernel(x, edge_index, W1, b1, Wl, bl, Wr, W2, b2):
    eye8 = jnp.eye(8, dtype=jnp.float32)
    w1d = jnp.kron(eye8, W1.T)
    b1p = jnp.tile(b1, 8).reshape(1, 128)
    xp = jnp.pad(x.reshape(_N // 8, 128), ((0, (_NA - _N) // 8), (0, 0)))

    hp = pl.pallas_call(
        _lin1_body,
        grid=(_GRIDP,),
        in_specs=[pl.BlockSpec((_BP, 128), lambda i: (i, 0)),
                  pl.BlockSpec((128, 128), lambda i: (0, 0)),
                  pl.BlockSpec((1, 128), lambda i: (0, 0))],
        out_specs=pl.BlockSpec((_BP, 128), lambda i: (i, 0)),
        out_shape=jax.ShapeDtypeStruct((_NP, 128), jnp.float32),
    )(xp, w1d, b1p)

    h = hp.reshape(_NA, 16)
    e3 = edge_index.reshape(2 * _E)
    z16 = jnp.zeros((_CW, 16), jnp.float32)
    z1 = jnp.zeros((_CW,), jnp.float32)
    ssum_p, cnt_p = _sc_agg(e3, h, z16, z1)

    sp = ssum_p.reshape(2 * _NP, 128)
    cexp = jnp.broadcast_to(cnt_p.reshape(2 * _NA, 1), (2 * _NA, 16))
    cp = cexp.reshape(2 * _NP, 128)

    wld = jnp.kron(eye8, Wl.T)
    wrd = jnp.kron(eye8, Wr.T)
    w2d = jnp.kron(eye8, W2.T)
    blp = jnp.tile(bl, 8).reshape(1, 128)
    b2p = jnp.tile(b2, 8).reshape(1, 256)

    outp = pl.pallas_call(
        _final_body,
        grid=(_GRIDP,),
        in_specs=[pl.BlockSpec((_BP, 128), lambda i: (i, 0)),
                  pl.BlockSpec((_BP, 128), lambda i: (i + _GRIDP, 0)),
                  pl.BlockSpec((_BP, 128), lambda i: (i, 0)),
                  pl.BlockSpec((_BP, 128), lambda i: (i + _GRIDP, 0)),
                  pl.BlockSpec((_BP, 128), lambda i: (i, 0)),
                  pl.BlockSpec((128, 128), lambda i: (0, 0)),
                  pl.BlockSpec((1, 128), lambda i: (0, 0)),
                  pl.BlockSpec((128, 128), lambda i: (0, 0)),
                  pl.BlockSpec((128, 256), lambda i: (0, 0)),
                  pl.BlockSpec((1, 256), lambda i: (0, 0))],
        out_specs=pl.BlockSpec((_BP, 256), lambda i: (i, 0)),
        out_shape=jax.ShapeDtypeStruct((_NP, 256), jnp.float32),
    )(sp, sp, cp, cp, hp, wld, blp, wrd, w2d, b2p)
    return outp[:_N // 8].reshape(_N, 32)


# speedup vs baseline: 1.1428x; 1.1081x over previous
import functools

import jax
import jax.numpy as jnp
from jax import lax
from jax.experimental import pallas as pl
from jax.experimental.pallas import tpu as pltpu
from jax.experimental.pallas import tpu_sc as plsc

_N = 100000
_E = 3200000
_CH = 128
_GROUP = 8
_NW = 32
_NGROUPS = _E // (_CH * _GROUP)
_NCHUNKS = _E // _CH
_NA = _N + 96
_CW = 6256
_NP = _NA // 8
_BP = 3128
_GRIDP = _NP // _BP


def _sc_agg_body(e3_hbm, h_hbm, z16_hbm, z1_hbm, ssum_out, cnt_out,
                 sidx, didx, rbuf, ones,
                 ssum_sh, cnt_sh,
                 g0, g1, g2, g3, g4, g5, g6, g7,
                 s0, s1, s2, s3, s4, s5, s6, s7, csem, isem):
    cid = lax.axis_index("c")
    sid = lax.axis_index("s")
    wid = cid * 16 + sid
    gsems = (g0, g1, g2, g3, g4, g5, g6, g7)
    ssems = (s0, s1, s2, s3, s4, s5, s6, s7)

    one16 = jnp.ones((16,), jnp.float32)

    def ofill(i, c):
        ones[pl.ds(i * 16, 16)] = one16
        return c
    lax.fori_loop(0, _GROUP * _CH // 16, ofill, 0)

    rbase = sid * _CW
    pltpu.sync_copy(z16_hbm, ssum_sh.at[pl.ds(rbase, _CW)])
    pltpu.sync_copy(z1_hbm, cnt_sh.at[pl.ds(rbase, _CW)])

    plsc.subcore_barrier()

    _GE = _GROUP * _CH

    def fire_idx(b, g):
        pltpu.async_copy(e3_hbm.at[pl.ds(g * _GE, _GE)], sidx.at[b], isem)
        pltpu.async_copy(e3_hbm.at[pl.ds(_E + g * _GE, _GE)], didx.at[b],
                         isem)

    def wait_idx(b):
        pltpu.make_async_copy(e3_hbm.at[pl.ds(0, _GE)], sidx.at[b],
                              isem).wait()
        pltpu.make_async_copy(e3_hbm.at[pl.ds(0, _GE)], didx.at[b],
                              isem).wait()

    def wait_slot(ob, j):
        dj = didx.at[ob, pl.ds(j * _CH, _CH)]
        pltpu.make_async_copy(rbuf.at[pl.ds(j * _CH, _CH)],
                              ssum_sh.at[dj], ssems[j]).wait()

    def drain_cnt(ob):
        for j in range(_GROUP):
            dj = didx.at[ob, pl.ds(j * _CH, _CH)]
            pltpu.make_async_copy(ones.at[pl.ds(0, _CH)], cnt_sh.at[dj],
                                  csem).wait()

    def run_group(b, guard, fire_next):
        wait_idx(b)
        for j in range(_GROUP):
            if guard is None:
                wait_slot(1 - b, j)
            else:
                @pl.when(guard)
                def _(j=j):
                    wait_slot(1 - b, j)
            pltpu.async_copy(h_hbm.at[sidx.at[b, pl.ds(j * _CH, _CH)]],
                             rbuf.at[pl.ds(j * _CH, _CH)], gsems[j])
        if guard is None:
            drain_cnt(1 - b)
        else:
            @pl.when(guard)
            def _():
                drain_cnt(1 - b)
        if fire_next is not None:
            fire_next()
        for j in range(_GROUP):
            dj = didx.at[b, pl.ds(j * _CH, _CH)]
            pltpu.make_async_copy(h_hbm.at[sidx.at[b, pl.ds(j * _CH, _CH)]],
                                  rbuf.at[pl.ds(j * _CH, _CH)],
                                  gsems[j]).wait()
            pltpu.async_copy(rbuf.at[pl.ds(j * _CH, _CH)], ssum_sh.at[dj],
                             ssems[j], add=True)
            pltpu.async_copy(ones.at[pl.ds(0, _CH)], cnt_sh.at[dj], csem,
                             add=True)

    fire_idx(0, wid)

    def pair_body(k2, c):
        k0 = 2 * k2
        run_group(0, k2 > 0,
                  lambda: fire_idx(1, wid + (k0 + 1) * _NW))
        run_group(1, None,
                  lambda: fire_idx(0, wid + (k0 + 2) * _NW))
        return c
    lax.fori_loop(0, 48, pair_body, 0)

    ntail = _NGROUPS - 97 * _NW

    def fire_97():
        @pl.when(wid < ntail)
        def _():
            fire_idx(1, wid + 97 * _NW)
    run_group(0, None, fire_97)

    @pl.when(wid < ntail)
    def _():
        run_group(1, None, None)

    for j in range(_GROUP):
        dj = didx.at[0, pl.ds(j * _CH, _CH)]
        pltpu.make_async_copy(rbuf.at[pl.ds(j * _CH, _CH)],
                              ssum_sh.at[dj], ssems[j]).wait()
        pltpu.make_async_copy(ones.at[pl.ds(0, _CH)], cnt_sh.at[dj],
                              csem).wait()

    plsc.subcore_barrier()

    orow = cid * _NA + rbase
    pltpu.sync_copy(ssum_sh.at[pl.ds(rbase, _CW)],
                    ssum_out.at[pl.ds(orow, _CW)])
    pltpu.sync_copy(cnt_sh.at[pl.ds(rbase, _CW)],
                    cnt_out.at[pl.ds(orow, _CW)])


_sc_agg = functools.partial(
    pl.kernel,
    out_type=(jax.ShapeDtypeStruct((2 * _NA, 16), jnp.float32),
              jax.ShapeDtypeStruct((2 * _NA,), jnp.float32)),
    mesh=plsc.VectorSubcoreMesh(core_axis_name="c", subcore_axis_name="s"),
    compiler_params=pltpu.CompilerParams(use_tc_tiling_on_sc=False),
    scratch_types=[
        pltpu.VMEM((2, _GROUP * _CH), jnp.int32),
        pltpu.VMEM((2, _GROUP * _CH), jnp.int32),
        pltpu.VMEM((_GROUP * _CH, 16), jnp.float32),
        pltpu.VMEM((_GROUP * _CH,), jnp.float32),
        pltpu.VMEM_SHARED((_NA, 16), jnp.float32),
        pltpu.VMEM_SHARED((_NA,), jnp.float32),
    ] + [pltpu.SemaphoreType.DMA] * 18,
)(_sc_agg_body)


def _lin1_body(x_ref, w_ref, b_ref, o_ref):
    o_ref[...] = jnp.maximum(
        jnp.dot(x_ref[...], w_ref[...], preferred_element_type=jnp.float32)
        + b_ref[...], 0.0)


def _final_body(s0, s1, c0, c1, h_ref, wl, bl, wr, w2, b2, o_ref):
    cnt = jnp.maximum(c0[...] + c1[...], 1.0)
    aggr = (s0[...] + s1[...]) / cnt
    h2 = jnp.maximum(
        jnp.dot(aggr, wl[...], preferred_element_type=jnp.float32) + bl[...]
        + jnp.dot(h_ref[...], wr[...], preferred_element_type=jnp.float32),
        0.0)
    o_ref[...] = (jnp.dot(h2, w2[...], preferred_element_type=jnp.float32)
                  + b2[...])


@jax.jit
def kernel(x, edge_index, W1, b1, Wl, bl, Wr, W2, b2):
    eye8 = jnp.eye(8, dtype=jnp.float32)
    w1d = jnp.kron(eye8, W1.T)
    b1p = jnp.tile(b1, 8).reshape(1, 128)
    xp = jnp.pad(x.reshape(_N // 8, 128), ((0, (_NA - _N) // 8), (0, 0)))

    hp = pl.pallas_call(
        _lin1_body,
        grid=(_GRIDP,),
        in_specs=[pl.BlockSpec((_BP, 128), lambda i: (i, 0)),
                  pl.BlockSpec((128, 128), lambda i: (0, 0)),
                  pl.BlockSpec((1, 128), lambda i: (0, 0))],
        out_specs=pl.BlockSpec((_BP, 128), lambda i: (i, 0)),
        out_shape=jax.ShapeDtypeStruct((_NP, 128), jnp.float32),
    )(xp, w1d, b1p)

    h = hp.reshape(_NA, 16)
    e3 = edge_index.reshape(2 * _E)
    z16 = jnp.zeros((_CW, 16), jnp.float32)
    z1 = jnp.zeros((_CW,), jnp.float32)
    ssum_p, cnt_p = _sc_agg(e3, h, z16, z1)

    sp = ssum_p.reshape(2 * _NP, 128)
    cexp = jnp.broadcast_to(cnt_p.reshape(2 * _NA, 1), (2 * _NA, 16))
    cp = cexp.reshape(2 * _NP, 128)

    wld = jnp.kron(eye8, Wl.T)
    wrd = jnp.kron(eye8, Wr.T)
    w2d = jnp.kron(eye8, W2.T)
    blp = jnp.tile(bl, 8).reshape(1, 128)
    b2p = jnp.tile(b2, 8).reshape(1, 256)

    outp = pl.pallas_call(
        _final_body,
        grid=(_GRIDP,),
        in_specs=[pl.BlockSpec((_BP, 128), lambda i: (i, 0)),
                  pl.BlockSpec((_BP, 128), lambda i: (i + _GRIDP, 0)),
                  pl.BlockSpec((_BP, 128), lambda i: (i, 0)),
                  pl.BlockSpec((_BP, 128), lambda i: (i + _GRIDP, 0)),
                  pl.BlockSpec((_BP, 128), lambda i: (i, 0)),
                  pl.BlockSpec((128, 128), lambda i: (0, 0)),
                  pl.BlockSpec((1, 128), lambda i: (0, 0)),
                  pl.BlockSpec((128, 128), lambda i: (0, 0)),
                  pl.BlockSpec((128, 256), lambda i: (0, 0)),
                  pl.BlockSpec((1, 256), lambda i: (0, 0))],
        out_specs=pl.BlockSpec((_BP, 256), lambda i: (i, 0)),
        out_shape=jax.ShapeDtypeStruct((_NP, 256), jnp.float32),
    )(sp, sp, cp, cp, hp, wld, blp, wrd, w2d, b2p)
    return outp[:_N // 8].reshape(_N, 32)
